# Initial kernel scaffold; baseline (speedup 1.0000x reference)
import jax
import jax.numpy as jnp
from jax.experimental import pallas as pl


def kernel(x, edge_index, edge_attr, W_lin, W1, b1, W2, b2):
    raise NotImplementedError("write your pallas kernel here")


# speedup vs baseline: 1.9256x; 1.9256x over previous
import functools

import jax
import jax.numpy as jnp
from jax import lax
from jax.experimental import pallas as pl
from jax.experimental.pallas import tpu as pltpu
from jax.experimental.pallas import tpu_sc as plsc

N = 10000
E = 160000
D = 256
DE = 16
H = 128
M_BLK = 1000
E_BLK = 2000
CH = 80
N_SUB = 16
EPT = E // N_SUB
N_CH = EPT // CH
NP = 10240
RPT = NP // N_SUB


def _prep_body(x_ref, wlt_ref, w1at_ref, h_ref, hw_ref):
    h = jnp.maximum(
        jnp.dot(x_ref[...], wlt_ref[...], preferred_element_type=jnp.float32), 0.0)
    h_ref[...] = h
    hw = jnp.dot(h, w1at_ref[...], preferred_element_type=jnp.float32)
    hw_ref[0] = hw[:, :H]
    hw_ref[1] = hw[:, H:]


def _prep(x, wlt, w1at):
    return pl.pallas_call(
        _prep_body,
        grid=(N // M_BLK,),
        in_specs=[
            pl.BlockSpec((M_BLK, D), lambda i: (i, 0)),
            pl.BlockSpec((D, D), lambda i: (0, 0)),
            pl.BlockSpec((D, D), lambda i: (0, 0)),
        ],
        out_specs=[
            pl.BlockSpec((M_BLK, D), lambda i: (i, 0)),
            pl.BlockSpec((2, M_BLK, H), lambda i: (0, i, 0)),
        ],
        out_shape=[
            jax.ShapeDtypeStruct((N, D), jnp.float32),
            jax.ShapeDtypeStruct((2, N, H), jnp.float32),
        ],
    )(x, wlt, w1at)


def _edge_body(ea_ref, w1bt_ref, b1_ref, ew_ref):
    ew = jnp.dot(ea_ref[...], w1bt_ref[...],
                 preferred_element_type=jnp.float32) + b1_ref[...]
    ew_ref[0] = ew[:, :H]
    ew_ref[1] = ew[:, H:]


def _edge(edge_attr, w1bt, b1):
    return pl.pallas_call(
        _edge_body,
        grid=(E // E_BLK,),
        in_specs=[
            pl.BlockSpec((E_BLK, DE), lambda i: (i, 0)),
            pl.BlockSpec((DE, D), lambda i: (0, 0)),
            pl.BlockSpec((1, D), lambda i: (0, 0)),
        ],
        out_specs=[pl.BlockSpec((2, E_BLK, H), lambda i: (0, i, 0))],
        out_shape=[jax.ShapeDtypeStruct((2, E, H), jnp.float32)],
    )(edge_attr, w1bt, b1)[0]


def _out_body(ns_ref, h_ref, w2t_ref, b2_ref, o_ref):
    acc = jnp.dot(ns_ref[0], w2t_ref[0:H, :], preferred_element_type=jnp.float32)
    acc = acc + jnp.dot(ns_ref[1], w2t_ref[H:2 * H, :],
                        preferred_element_type=jnp.float32)
    acc = acc + jnp.dot(h_ref[...], w2t_ref[2 * H:, :],
                        preferred_element_type=jnp.float32)
    o_ref[...] = jnp.maximum(acc + b2_ref[...], 0.0)


def _out(ns_s, h, w2t, b2):
    return pl.pallas_call(
        _out_body,
        grid=(N // M_BLK,),
        in_specs=[
            pl.BlockSpec((2, M_BLK, H), lambda i: (0, i, 0)),
            pl.BlockSpec((M_BLK, D), lambda i: (i, 0)),
            pl.BlockSpec((2 * D, D), lambda i: (0, 0)),
            pl.BlockSpec((1, D), lambda i: (0, 0)),
        ],
        out_specs=pl.BlockSpec((M_BLK, D), lambda i: (i, 0)),
        out_shape=jax.ShapeDtypeStruct((N, D), jnp.float32),
    )(ns_s, h, w2t, b2)


@functools.cache
def _get_sc_kernel():
    mesh = plsc.VectorSubcoreMesh(core_axis_name="c", subcore_axis_name="s")

    @functools.partial(
        pl.kernel,
        mesh=mesh,
        out_type=jax.ShapeDtypeStruct((2 * NP, H), jnp.float32),
        scratch_types=[
            pltpu.VMEM((CH,), jnp.int32),
            pltpu.VMEM((CH,), jnp.int32),
            pltpu.VMEM((CH, H), jnp.float32),
            pltpu.VMEM((CH, H), jnp.float32),
            pltpu.VMEM_SHARED((NP, H), jnp.float32),
            pltpu.SemaphoreType.DMA,
        ],
    )
    def _sc_edge_agg(hw_hbm, ew_hbm, src2_hbm, dst_hbm, zeros_hbm,
                     out_hbm, sidx_v, didx_v, gbuf, ebuf, accum, sem):
        _sc_body(hw_hbm, ew_hbm, src2_hbm, dst_hbm, zeros_hbm,
                 out_hbm, sidx_v, didx_v, gbuf, ebuf, accum, sem)

    return _sc_edge_agg


def _sc_body(hw_hbm, ew_hbm, src2_hbm, dst_hbm, zeros_hbm,
             out_hbm, sidx_v, didx_v, gbuf, ebuf, accum, sem):
    c = lax.axis_index("c")
    s = lax.axis_index("s")
    pltpu.sync_copy(zeros_hbm.at[pl.ds(s * RPT, RPT)],
                    accum.at[pl.ds(s * RPT, RPT)])
    plsc.subcore_barrier()
    ebase2 = c * E + s * EPT

    def chunk(k, carry):
        base = s * EPT + k * CH
        base2 = ebase2 + k * CH
        pltpu.sync_copy(src2_hbm.at[pl.ds(base2, CH)], sidx_v)
        pltpu.sync_copy(dst_hbm.at[pl.ds(base, CH)], didx_v)
        pltpu.async_copy(hw_hbm.at[sidx_v], gbuf, sem).wait()
        pltpu.sync_copy(ew_hbm.at[pl.ds(base2, CH)], ebuf)

        def row(r, rc):
            for j in range(H // 16):
                sl = pl.ds(j * 16, 16)
                gbuf[r, sl] = jnp.maximum(gbuf[r, sl] + ebuf[r, sl], 0.0)
            return rc
        lax.fori_loop(0, CH, row, 0)
        pltpu.sync_copy(gbuf, accum.at[didx_v], add=True)
        return carry

    lax.fori_loop(0, N_CH, chunk, 0)
    plsc.subcore_barrier()
    pltpu.sync_copy(accum.at[pl.ds(s * RPT, RPT)],
                    out_hbm.at[pl.ds(c * NP + s * RPT, RPT)])


def kernel(x, edge_index, edge_attr, W_lin, W1, b1, W2, b2):
    src = edge_index[0].astype(jnp.int32)
    dst = edge_index[1].astype(jnp.int32)
    src2 = jnp.concatenate([src, src + N])
    wlt = W_lin.T
    w1at = W1[:, :D].T
    w1bt = W1[:, D:].T
    w2t = W2.T
    h, hw_s = _prep(x, wlt, w1at)
    ew_s = _edge(edge_attr, w1bt, b1.reshape(1, D))
    hw_flat = hw_s.reshape(2 * N, H)
    ew_flat = ew_s.reshape(2 * E, H)
    zeros = jnp.zeros((NP, H), jnp.float32)
    ns_flat = _get_sc_kernel()(hw_flat, ew_flat, src2, dst, zeros)
    ns_s = ns_flat.reshape(2, NP, H)
    return _out(ns_s, h, w2t, b2.reshape(1, D))


# speedup vs baseline: 2.3243x; 1.2071x over previous
import functools

import jax
import jax.numpy as jnp
from jax import lax
from jax.experimental import pallas as pl
from jax.experimental.pallas import tpu as pltpu
from jax.experimental.pallas import tpu_sc as plsc

N = 10000
E = 160000
D = 256
DE = 16
H = 128
M_BLK = 1000
E_BLK = 2048
CH = 32
N_SUB = 16
EP = 163840
EPT = EP // N_SUB
N_CH = EPT // CH
NP = 10240
RPT = NP // N_SUB


def _prep_body(x_ref, wlt_ref, w1at_ref, h_ref, hw_ref):
    h = jnp.maximum(
        jnp.dot(x_ref[...], wlt_ref[...], preferred_element_type=jnp.float32), 0.0)
    h_ref[...] = h
    hw = jnp.dot(h, w1at_ref[...], preferred_element_type=jnp.float32)
    hw_ref[0] = hw[:, :H]
    hw_ref[1] = hw[:, H:]


def _prep(x, wlt, w1at):
    return pl.pallas_call(
        _prep_body,
        grid=(N // M_BLK,),
        in_specs=[
            pl.BlockSpec((M_BLK, D), lambda i: (i, 0)),
            pl.BlockSpec((D, D), lambda i: (0, 0)),
            pl.BlockSpec((D, D), lambda i: (0, 0)),
        ],
        out_specs=[
            pl.BlockSpec((M_BLK, D), lambda i: (i, 0)),
            pl.BlockSpec((2, M_BLK, H), lambda i: (0, i, 0)),
        ],
        out_shape=[
            jax.ShapeDtypeStruct((N, D), jnp.float32),
            jax.ShapeDtypeStruct((2, N, H), jnp.float32),
        ],
    )(x, wlt, w1at)


def _edge_body(ea_ref, w1bt_ref, b1_ref, ew_ref):
    ew = jnp.dot(ea_ref[...], w1bt_ref[...],
                 preferred_element_type=jnp.float32) + b1_ref[...]
    ew_ref[0] = ew[:, :H]
    ew_ref[1] = ew[:, H:]


def _edge(edge_attr, w1bt, b1):
    return pl.pallas_call(
        _edge_body,
        grid=(EP // E_BLK,),
        in_specs=[
            pl.BlockSpec((E_BLK, DE), lambda i: (i, 0)),
            pl.BlockSpec((DE, D), lambda i: (0, 0)),
            pl.BlockSpec((1, D), lambda i: (0, 0)),
        ],
        out_specs=[pl.BlockSpec((2, E_BLK, H), lambda i: (0, i, 0))],
        out_shape=[jax.ShapeDtypeStruct((2, EP, H), jnp.float32)],
    )(edge_attr, w1bt, b1)[0]


def _out_body(ns_ref, h_ref, w2t_ref, b2_ref, o_ref):
    acc = jnp.dot(ns_ref[0], w2t_ref[0:H, :], preferred_element_type=jnp.float32)
    acc = acc + jnp.dot(ns_ref[1], w2t_ref[H:2 * H, :],
                        preferred_element_type=jnp.float32)
    acc = acc + jnp.dot(h_ref[...], w2t_ref[2 * H:, :],
                        preferred_element_type=jnp.float32)
    o_ref[...] = jnp.maximum(acc + b2_ref[...], 0.0)


def _out(ns_s, h, w2t, b2):
    return pl.pallas_call(
        _out_body,
        grid=(N // M_BLK,),
        in_specs=[
            pl.BlockSpec((2, M_BLK, H), lambda i: (0, i, 0)),
            pl.BlockSpec((M_BLK, D), lambda i: (i, 0)),
            pl.BlockSpec((2 * D, D), lambda i: (0, 0)),
            pl.BlockSpec((1, D), lambda i: (0, 0)),
        ],
        out_specs=pl.BlockSpec((M_BLK, D), lambda i: (i, 0)),
        out_shape=jax.ShapeDtypeStruct((N, D), jnp.float32),
    )(ns_s, h, w2t, b2)


@functools.cache
def _get_sc_kernel():
    mesh = plsc.VectorSubcoreMesh(core_axis_name="c", subcore_axis_name="s")

    @functools.partial(
        pl.kernel,
        mesh=mesh,
        out_type=jax.ShapeDtypeStruct((2 * NP, H), jnp.float32),
        scratch_types=[
            pltpu.VMEM((EPT,), jnp.int32),
            pltpu.VMEM((EPT,), jnp.int32),
            pltpu.VMEM((CH,), jnp.int32),
            pltpu.VMEM((CH,), jnp.int32),
            pltpu.VMEM((CH,), jnp.int32),
            pltpu.VMEM((CH,), jnp.int32),
            pltpu.VMEM((CH, H), jnp.float32),
            pltpu.VMEM((CH, H), jnp.float32),
            pltpu.VMEM((CH, H), jnp.float32),
            pltpu.VMEM((CH, H), jnp.float32),
            pltpu.VMEM_SHARED((NP, H), jnp.float32),
            pltpu.SemaphoreType.DMA,
            pltpu.SemaphoreType.DMA,
            pltpu.SemaphoreType.DMA,
            pltpu.SemaphoreType.DMA,
        ],
    )
    def _sc_edge_agg(hw_hbm, ew_hbm, src2_hbm, dst_hbm, zeros_hbm, out_hbm,
                     srcall, dstall, sidx0, sidx1, didx0, didx1,
                     g0, g1, e0, e1, accum, semg0, semg1, seme0, seme1):
        _sc_body(hw_hbm, ew_hbm, src2_hbm, dst_hbm, zeros_hbm, out_hbm,
                 srcall, dstall, sidx0, sidx1, didx0, didx1,
                 g0, g1, e0, e1, accum, semg0, semg1, seme0, seme1)

    return _sc_edge_agg


def _sc_body(hw_hbm, ew_hbm, src2_hbm, dst_hbm, zeros_hbm, out_hbm,
             srcall, dstall, sidx0, sidx1, didx0, didx1,
             g0, g1, e0, e1, accum, semg0, semg1, seme0, seme1):
    c = lax.axis_index("c")
    s = lax.axis_index("s")
    ebase2 = c * EP + s * EPT
    pltpu.sync_copy(zeros_hbm.at[pl.ds(s * RPT, RPT)],
                    accum.at[pl.ds(s * RPT, RPT)])
    pltpu.sync_copy(src2_hbm.at[pl.ds(ebase2, EPT)], srcall)
    pltpu.sync_copy(dst_hbm.at[pl.ds(s * EPT, EPT)], dstall)
    plsc.subcore_barrier()

    bufs = ((sidx0, didx0, g0, e0, semg0, seme0),
            (sidx1, didx1, g1, e1, semg1, seme1))

    def fill_idx(b, k):
        sidx, didx = b[0], b[1]
        off = k * CH
        for j in range(CH // 16):
            sl = pl.ds(j * 16, 16)
            sidx[sl] = srcall[pl.ds(off + j * 16, 16)]
            didx[sl] = dstall[pl.ds(off + j * 16, 16)]

    def start_fetch(b, k):
        pltpu.async_copy(hw_hbm.at[b[0]], b[2], b[4])
        pltpu.async_copy(ew_hbm.at[pl.ds(ebase2 + k * CH, CH)], b[3], b[5])

    def process(bs, other, k):
        @pl.when(k + 1 < N_CH)
        def _():
            fill_idx(other, k + 1)
            start_fetch(other, k + 1)
        g, eb = bs[2], bs[3]
        pltpu.make_async_copy(hw_hbm.at[bs[0]], g, bs[4]).wait()
        pltpu.make_async_copy(ew_hbm.at[pl.ds(ebase2 + k * CH, CH)],
                              eb, bs[5]).wait()

        def row(r, rc):
            for j in range(H // 16):
                sl = pl.ds(j * 16, 16)
                g[r, sl] = jnp.maximum(g[r, sl] + eb[r, sl], 0.0)
            return rc
        lax.fori_loop(0, CH, row, 0)
        pltpu.sync_copy(g, accum.at[bs[1]], add=True)

    fill_idx(bufs[0], 0)
    start_fetch(bufs[0], 0)

    def chunk(k, carry):
        @pl.when(k % 2 == 0)
        def _():
            process(bufs[0], bufs[1], k)

        @pl.when(k % 2 == 1)
        def _():
            process(bufs[1], bufs[0], k)
        return carry

    lax.fori_loop(0, N_CH, chunk, 0)
    plsc.subcore_barrier()
    pltpu.sync_copy(accum.at[pl.ds(s * RPT, RPT)],
                    out_hbm.at[pl.ds(c * NP + s * RPT, RPT)])


def kernel(x, edge_index, edge_attr, W_lin, W1, b1, W2, b2):
    src = edge_index[0].astype(jnp.int32)
    dst = edge_index[1].astype(jnp.int32)
    srcp = jnp.concatenate([src, jnp.zeros((EP - E,), jnp.int32)])
    dstp = jnp.concatenate([dst, jnp.full((EP - E,), N, jnp.int32)])
    src2 = jnp.concatenate([srcp, srcp + N])
    wlt = W_lin.T
    w1at = W1[:, :D].T
    w1bt = W1[:, D:].T
    w2t = W2.T
    h, hw_s = _prep(x, wlt, w1at)
    ea_p = jnp.concatenate(
        [edge_attr, jnp.zeros((EP - E, DE), jnp.float32)])
    ew_s = _edge(ea_p, w1bt, b1.reshape(1, D))
    hw_flat = hw_s.reshape(2 * N, H)
    ew_flat = ew_s.reshape(2 * EP, H)
    zeros = jnp.zeros((NP, H), jnp.float32)
    ns_flat = _get_sc_kernel()(hw_flat, ew_flat, src2, dstp, zeros)
    ns_s = ns_flat.reshape(2, NP, H)
    return _out(ns_s, h, w2t, b2.reshape(1, D))


# speedup vs baseline: 2.4498x; 1.0540x over previous
import functools

import jax
import jax.numpy as jnp
from jax import lax
from jax.experimental import pallas as pl
from jax.experimental.pallas import tpu as pltpu
from jax.experimental.pallas import tpu_sc as plsc

N = 10000
E = 160000
D = 256
DE = 16
H = 128
M_BLK = 1000
E_BLK = 2048
CH = 64
N_SUB = 16
EP = 163840
EPT = EP // N_SUB
N_CH = EPT // CH
NP = 10240
RPT = NP // N_SUB


def _prep_body(x_ref, wlt_ref, w1at_ref, h_ref, hw_ref):
    h = jnp.maximum(
        jnp.dot(x_ref[...], wlt_ref[...], preferred_element_type=jnp.float32), 0.0)
    h_ref[...] = h
    hw = jnp.dot(h, w1at_ref[...], preferred_element_type=jnp.float32)
    hw_ref[0] = hw[:, :H]
    hw_ref[1] = hw[:, H:]


def _prep(x, wlt, w1at):
    return pl.pallas_call(
        _prep_body,
        grid=(N // M_BLK,),
        in_specs=[
            pl.BlockSpec((M_BLK, D), lambda i: (i, 0)),
            pl.BlockSpec((D, D), lambda i: (0, 0)),
            pl.BlockSpec((D, D), lambda i: (0, 0)),
        ],
        out_specs=[
            pl.BlockSpec((M_BLK, D), lambda i: (i, 0)),
            pl.BlockSpec((2, M_BLK, H), lambda i: (0, i, 0)),
        ],
        out_shape=[
            jax.ShapeDtypeStruct((N, D), jnp.float32),
            jax.ShapeDtypeStruct((2, N, H), jnp.float32),
        ],
    )(x, wlt, w1at)


def _edge_body(ea_ref, w1bt_ref, b1_ref, ew_ref):
    ew = jnp.dot(ea_ref[...], w1bt_ref[...],
                 preferred_element_type=jnp.float32) + b1_ref[...]
    ew_ref[0] = ew[:, :H]
    ew_ref[1] = ew[:, H:]


def _edge(edge_attr, w1bt, b1):
    return pl.pallas_call(
        _edge_body,
        grid=(EP // E_BLK,),
        in_specs=[
            pl.BlockSpec((E_BLK, DE), lambda i: (i, 0)),
            pl.BlockSpec((DE, D), lambda i: (0, 0)),
            pl.BlockSpec((1, D), lambda i: (0, 0)),
        ],
        out_specs=[pl.BlockSpec((2, E_BLK, H), lambda i: (0, i, 0))],
        out_shape=[jax.ShapeDtypeStruct((2, EP, H), jnp.float32)],
    )(edge_attr, w1bt, b1)[0]


def _out_body(ns_ref, h_ref, w2t_ref, b2_ref, o_ref):
    acc = jnp.dot(ns_ref[0], w2t_ref[0:H, :], preferred_element_type=jnp.float32)
    acc = acc + jnp.dot(ns_ref[1], w2t_ref[H:2 * H, :],
                        preferred_element_type=jnp.float32)
    acc = acc + jnp.dot(h_ref[...], w2t_ref[2 * H:, :],
                        preferred_element_type=jnp.float32)
    o_ref[...] = jnp.maximum(acc + b2_ref[...], 0.0)


def _out(ns_s, h, w2t, b2):
    return pl.pallas_call(
        _out_body,
        grid=(N // M_BLK,),
        in_specs=[
            pl.BlockSpec((2, M_BLK, H), lambda i: (0, i, 0)),
            pl.BlockSpec((M_BLK, D), lambda i: (i, 0)),
            pl.BlockSpec((2 * D, D), lambda i: (0, 0)),
            pl.BlockSpec((1, D), lambda i: (0, 0)),
        ],
        out_specs=pl.BlockSpec((M_BLK, D), lambda i: (i, 0)),
        out_shape=jax.ShapeDtypeStruct((N, D), jnp.float32),
    )(ns_s, h, w2t, b2)


@functools.cache
def _get_sc_kernel():
    mesh = plsc.VectorSubcoreMesh(core_axis_name="c", subcore_axis_name="s")

    @functools.partial(
        pl.kernel,
        mesh=mesh,
        out_type=jax.ShapeDtypeStruct((2 * NP, H), jnp.float32),
        scratch_types=[
            pltpu.VMEM((CH,), jnp.int32),
            pltpu.VMEM((CH,), jnp.int32),
            pltpu.VMEM((CH,), jnp.int32),
            pltpu.VMEM((CH,), jnp.int32),
            pltpu.VMEM((CH,), jnp.int32),
            pltpu.VMEM((CH,), jnp.int32),
            pltpu.VMEM((CH,), jnp.int32),
            pltpu.VMEM((CH,), jnp.int32),
            pltpu.VMEM((CH, H), jnp.float32),
            pltpu.VMEM((CH, H), jnp.float32),
            pltpu.VMEM((CH, H), jnp.float32),
            pltpu.VMEM((CH, H), jnp.float32),
            pltpu.VMEM_SHARED((NP, H), jnp.float32),
            pltpu.SemaphoreType.DMA,
            pltpu.SemaphoreType.DMA,
            pltpu.SemaphoreType.DMA,
            pltpu.SemaphoreType.DMA,
            pltpu.SemaphoreType.DMA,
            pltpu.SemaphoreType.DMA,
            pltpu.SemaphoreType.DMA,
            pltpu.SemaphoreType.DMA,
        ],
    )
    def _sc_edge_agg(hw_hbm, ew_hbm, src2_hbm, dst_hbm, zeros_hbm, out_hbm,
                     s0, s1, s2, s3, d0, d1, d2, d3,
                     g0, g1, e0, e1, accum,
                     si0, si1, si2, si3, sg0, sg1, se0, se1):
        _sc_body(hw_hbm, ew_hbm, src2_hbm, dst_hbm, zeros_hbm, out_hbm,
                 s0, s1, s2, s3, d0, d1, d2, d3,
                 g0, g1, e0, e1, accum,
                 si0, si1, si2, si3, sg0, sg1, se0, se1)

    return _sc_edge_agg


def _sc_body(hw_hbm, ew_hbm, src2_hbm, dst_hbm, zeros_hbm, out_hbm,
             s0, s1, s2, s3, d0, d1, d2, d3,
             g0, g1, e0, e1, accum,
             si0, si1, si2, si3, sg0, sg1, se0, se1):
    c = lax.axis_index("c")
    s = lax.axis_index("s")
    ebase2 = c * EP + s * EPT
    pltpu.sync_copy(zeros_hbm.at[pl.ds(s * RPT, RPT)],
                    accum.at[pl.ds(s * RPT, RPT)])
    plsc.subcore_barrier()

    isets = ((s0, d0, si0), (s1, d1, si1), (s2, d2, si2), (s3, d3, si3))
    dsets = ((g0, e0, sg0, se0), (g1, e1, sg1, se1))

    def start_idx(iset, k):
        off = k * CH
        pltpu.async_copy(src2_hbm.at[pl.ds(ebase2 + off, CH)], iset[0], iset[2])
        pltpu.async_copy(dst_hbm.at[pl.ds(s * EPT + off, CH)], iset[1], iset[2])

    def wait_idx(iset, k):
        off = k * CH
        pltpu.make_async_copy(src2_hbm.at[pl.ds(ebase2 + off, CH)],
                              iset[0], iset[2]).wait()
        pltpu.make_async_copy(dst_hbm.at[pl.ds(s * EPT + off, CH)],
                              iset[1], iset[2]).wait()

    def start_fetch(iset, dset, k):
        pltpu.async_copy(hw_hbm.at[iset[0]], dset[0], dset[2])
        pltpu.async_copy(ew_hbm.at[pl.ds(ebase2 + k * CH, CH)], dset[1], dset[3])

    def process(ia, ib, ic, da, db, k):
        @pl.when(k + 1 < N_CH)
        def _():
            wait_idx(ib, k + 1)
            start_fetch(ib, db, k + 1)

        @pl.when(k + 2 < N_CH)
        def _():
            start_idx(ic, k + 2)
        g, eb = da[0], da[1]
        pltpu.make_async_copy(hw_hbm.at[ia[0]], g, da[2]).wait()
        pltpu.make_async_copy(ew_hbm.at[pl.ds(ebase2 + k * CH, CH)],
                              eb, da[3]).wait()

        def row(r, rc):
            for j in range(H // 16):
                sl = pl.ds(j * 16, 16)
                g[r, sl] = jnp.maximum(g[r, sl] + eb[r, sl], 0.0)
            return rc
        lax.fori_loop(0, CH, row, 0)
        pltpu.sync_copy(g, accum.at[ia[1]], add=True)

    pltpu.sync_copy(src2_hbm.at[pl.ds(ebase2, CH)], s0)
    pltpu.sync_copy(dst_hbm.at[pl.ds(s * EPT, CH)], d0)
    start_fetch(isets[0], dsets[0], 0)
    start_idx(isets[1], 1)

    def chunk(k, carry):
        for m in range(4):
            @pl.when(k % 4 == m)
            def _(m=m):
                process(isets[m], isets[(m + 1) % 4], isets[(m + 2) % 4],
                        dsets[m % 2], dsets[(m + 1) % 2], k)
        return carry

    lax.fori_loop(0, N_CH, chunk, 0)
    plsc.subcore_barrier()
    pltpu.sync_copy(accum.at[pl.ds(s * RPT, RPT)],
                    out_hbm.at[pl.ds(c * NP + s * RPT, RPT)])


def kernel(x, edge_index, edge_attr, W_lin, W1, b1, W2, b2):
    src = edge_index[0].astype(jnp.int32)
    dst = edge_index[1].astype(jnp.int32)
    srcp = jnp.concatenate([src, jnp.zeros((EP - E,), jnp.int32)])
    dstp = jnp.concatenate([dst, jnp.full((EP - E,), N, jnp.int32)])
    src2 = jnp.concatenate([srcp, srcp + N])
    wlt = W_lin.T
    w1at = W1[:, :D].T
    w1bt = W1[:, D:].T
    w2t = W2.T
    h, hw_s = _prep(x, wlt, w1at)
    ea_p = jnp.concatenate(
        [edge_attr, jnp.zeros((EP - E, DE), jnp.float32)])
    ew_s = _edge(ea_p, w1bt, b1.reshape(1, D))
    hw_flat = hw_s.reshape(2 * N, H)
    ew_flat = ew_s.reshape(2 * EP, H)
    zeros = jnp.zeros((NP, H), jnp.float32)
    ns_flat = _get_sc_kernel()(hw_flat, ew_flat, src2, dstp, zeros)
    ns_s = ns_flat.reshape(2, NP, H)
    return _out(ns_s, h, w2t, b2.reshape(1, D))
